# baseline (speedup 1.0000x reference)
import functools

import jax
import jax.numpy as jnp
from jax import lax
from jax.experimental import pallas as pl
from jax.experimental.pallas import tpu as pltpu
from jax.experimental.pallas import tpu_sc as plsc

N_TOK = 2048
IN_DIM = 768
HID = 32768
K = 64

_NC = 2
_NS = 16
_NW = _NC * _NS
_RPW = N_TOK // _NW
_NBINS = 256


_H_BLK = 1024


def _enc_body(x_ref, w_ref, b_ref, out_ref):
    acc = lax.dot_general(
        x_ref[...], w_ref[...],
        (((1,), (1,)), ((), ())),
        preferred_element_type=jnp.float32,
    )
    out_ref[...] = jnp.maximum(acc + b_ref[...], 0.0)


def _encoder(x, W_enc, b_enc2d):
    ntok = x.shape[0]
    grid = (HID // _H_BLK,)
    return pl.pallas_call(
        _enc_body,
        grid=grid,
        in_specs=[
            pl.BlockSpec((ntok, IN_DIM), lambda h: (0, 0)),
            pl.BlockSpec((_H_BLK, IN_DIM), lambda h: (h, 0)),
            pl.BlockSpec((1, _H_BLK), lambda h: (0, h)),
        ],
        out_specs=pl.BlockSpec((ntok, _H_BLK), lambda h: (0, h)),
        out_shape=jax.ShapeDtypeStruct((ntok, HID), jnp.float32),
    )(x, W_enc, b_enc2d)


_UNROLL = 4


def _zero_hist(hist_ref, ncopy):
    zeros = jnp.zeros((16,), jnp.int32)

    @plsc.parallel_loop(0, _NBINS * ncopy // 16, unroll=8)
    def _(i):
        hist_ref[pl.ds(i * 16, 16)] = zeros


def _suffix_select(hist_ref, need, ncopy):
    def body(jj, st):
        carry, cnt_ge, above = st
        j = 15 - jj
        hv = jnp.zeros((16,), jnp.int32)
        for l in range(ncopy):
            hv = hv + hist_ref[pl.ds(l * _NBINS + j * 16, 16)]
        a = jnp.flip(jnp.cumsum(jnp.flip(hv, 0)), 0) + carry
        ge = a >= need
        cnt_ge = cnt_ge + jnp.sum(ge.astype(jnp.int32))
        above = jnp.maximum(above, jnp.max(jnp.where(ge, 0, a)))
        return (carry + jnp.sum(hv), cnt_ge, above)

    z = jnp.int32(0)
    _, cnt_ge, above = lax.fori_loop(0, 16, body, (z, z, z))
    return cnt_ge - 1, above


def _hist_full(src_ref, nvec, shift, maskval, hist_ref, lane_base, ones):
    @plsc.parallel_loop(0, nvec, unroll=16)
    def _(i):
        bits = jnp.maximum(src_ref[pl.ds(i * 16, 16)], 0)
        b = (lax.shift_right_logical(bits, shift) & maskval) + (
            lane_base + (i & (_UNROLL - 1)) * (_NBINS * 16))
        plsc.addupdate_scatter(hist_ref, [b], ones)


def _hist_masked(src_ref, n, shift, maskval, hist_ref, lane_base, ones):
    iota = lax.iota(jnp.int32, 16)

    @plsc.parallel_loop(0, (n + 15) // 16, unroll=4)
    def _(i):
        bits = jnp.maximum(src_ref[pl.ds(i * 16, 16)], 0)
        b = (lax.shift_right_logical(bits, shift) & maskval) + lane_base
        lanes_ok = (i * 16 + iota) < n
        plsc.addupdate_scatter(hist_ref, [b], ones, mask=lanes_ok)


def _compress_full(src_ref, nvec, shift, maskval, bsel, dst_ref):
    @plsc.parallel_loop(0, nvec, unroll=16, carry=jnp.int32(0))
    def off(i, off):
        v = src_ref[pl.ds(i * 16, 16)]
        b = lax.shift_right_logical(jnp.maximum(v, 0), shift) & maskval
        m = b == bsel
        plsc.store_compressed(dst_ref.at[pl.ds(off, 16)], v, mask=m)
        return off + jnp.sum(m.astype(jnp.int32))

    return off


def _compress_masked(src_ref, n, shift, maskval, bsel, dst_ref):
    iota = lax.iota(jnp.int32, 16)

    @plsc.parallel_loop(0, (n + 15) // 16, unroll=4, carry=jnp.int32(0))
    def off(i, off):
        v = src_ref[pl.ds(i * 16, 16)]
        b = lax.shift_right_logical(jnp.maximum(v, 0), shift) & maskval
        m = (b == bsel) & ((i * 16 + iota) < n)
        plsc.store_compressed(dst_ref.at[pl.ds(off, 16)], v, mask=m)
        return off + jnp.sum(m.astype(jnp.int32))

    return off


def _row_threshold(row_ref, cand_ref, hist_ref, lane_base, ones):
    need = jnp.int32(K)
    _zero_hist(hist_ref, 16 * _UNROLL)
    _hist_full(row_ref, HID // 16, 23, 0xFF, hist_ref, lane_base, ones)
    b1, above = _suffix_select(hist_ref, need, 16 * _UNROLL)
    need = need - above
    tbits = b1 << 23
    ncand = _compress_full(row_ref, HID // 16, 23, 0xFF, b1, cand_ref)
    for shift, width in ((15, 8), (7, 8), (0, 7)):
        maskval = (1 << width) - 1
        _zero_hist(hist_ref, 16)
        _hist_masked(cand_ref, ncand, shift, maskval, hist_ref, lane_base, ones)
        bk, above = _suffix_select(hist_ref, need, 16)
        need = need - above
        tbits = tbits | (bk << shift)
        if shift != 0:
            ncand = _compress_masked(cand_ref, ncand, shift, maskval, bk,
                                     cand_ref)
    return tbits


def _thresholds_sc(features_flat):
    ntok = features_flat.shape[0] // HID
    rpw = ntok // _NW
    mesh = plsc.VectorSubcoreMesh(
        core_axis_name="c", subcore_axis_name="s",
        num_cores=_NC, num_subcores=_NS)

    @functools.partial(
        pl.kernel,
        out_type=jax.ShapeDtypeStruct((ntok,), jnp.int32),
        mesh=mesh,
        compiler_params=pltpu.CompilerParams(needs_layout_passes=False),
        scratch_types=[
            pltpu.VMEM((HID,), jnp.int32),
            pltpu.VMEM((HID,), jnp.int32),
            pltpu.VMEM((HID + 16,), jnp.int32),
            pltpu.VMEM((_NBINS * 16 * _UNROLL,), jnp.int32),
            pltpu.VMEM((rpw,), jnp.int32),
            pltpu.SemaphoreType.DMA,
            pltpu.SemaphoreType.DMA,
        ],
    )
    def k(feat_hbm, out_hbm, row_a, row_b, cand_v, hist_v, thr_v, sem_a, sem_b):
        wid = lax.axis_index("s") * _NC + lax.axis_index("c")
        iota = lax.iota(jnp.int32, 16)
        lane_base = iota * _NBINS
        ones = jnp.ones((16,), jnp.int32)
        base_row = wid * rpw
        ngrp = rpw // 16

        def dma(r, buf, sem):
            return pltpu.make_async_copy(
                feat_hbm.at[pl.ds((base_row + r) * HID, HID)], buf, sem)

        def merge(tvecs, r, t):
            lane = r & 15
            grp = r >> 4
            return tuple(
                jnp.where((iota == lane) & (grp == g), t, tv)
                for g, tv in enumerate(tvecs))

        dma(0, row_a, sem_a).start()

        def pair_body(p, tvecs):
            ra = 2 * p
            dma(ra, row_a, sem_a).wait()
            dma(ra + 1, row_b, sem_b).start()
            t_a = _row_threshold(row_a, cand_v, hist_v, lane_base, ones)
            tvecs = merge(tvecs, ra, t_a)
            dma(ra + 1, row_b, sem_b).wait()

            @pl.when(p < rpw // 2 - 1)
            def _():
                dma(ra + 2, row_a, sem_a).start()

            t_b = _row_threshold(row_b, cand_v, hist_v, lane_base, ones)
            return merge(tvecs, ra + 1, t_b)

        z16 = jnp.zeros((16,), jnp.int32)
        tvecs = lax.fori_loop(0, rpw // 2, pair_body,
                              tuple(z16 for _ in range(ngrp)))
        for g in range(ngrp):
            thr_v[pl.ds(g * 16, 16)] = tvecs[g]
        pltpu.sync_copy(thr_v, out_hbm.at[pl.ds(base_row, rpw)])

    return k(features_flat)


_T_BLK = 256
_HC_BLK = 2048


def _dec_body(feat_ref, thr_ref, wdec_ref, sparse_ref, recon_ref, acc_ref):
    h = pl.program_id(0)
    t = pl.program_id(1)
    nh = pl.num_programs(0)
    f = feat_ref[...]
    tv = thr_ref[...]
    s = jnp.where((f >= tv) & (f > 0.0), f, 0.0)
    sparse_ref[...] = s
    part = lax.dot_general(
        s.astype(jnp.bfloat16), wdec_ref[...].astype(jnp.bfloat16),
        (((1,), (1,)), ((), ())),
        preferred_element_type=jnp.float32,
    )
    rows = pl.ds(t * _T_BLK, _T_BLK)

    @pl.when(h == 0)
    def _():
        acc_ref[rows, :] = part

    @pl.when(h > 0)
    def _():
        acc_ref[rows, :] = acc_ref[rows, :] + part

    @pl.when(h == nh - 1)
    def _():
        recon_ref[...] = acc_ref[rows, :]


def _mask_and_decode(features, thresholds2d, W_dec):
    grid = (HID // _HC_BLK, N_TOK // _T_BLK)
    return pl.pallas_call(
        _dec_body,
        grid=grid,
        in_specs=[
            pl.BlockSpec((_T_BLK, _HC_BLK), lambda h, t: (t, h)),
            pl.BlockSpec((_T_BLK, 1), lambda h, t: (t, 0)),
            pl.BlockSpec((IN_DIM, _HC_BLK), lambda h, t: (0, h)),
        ],
        out_specs=[
            pl.BlockSpec((_T_BLK, _HC_BLK), lambda h, t: (t, h)),
            pl.BlockSpec((_T_BLK, IN_DIM), lambda h, t: (t, 0)),
        ],
        out_shape=[
            jax.ShapeDtypeStruct((N_TOK, HID), jnp.float32),
            jax.ShapeDtypeStruct((N_TOK, IN_DIM), jnp.float32),
        ],
        scratch_shapes=[pltpu.VMEM((N_TOK, IN_DIM), jnp.float32)],
    )(features, thresholds2d, W_dec)


def kernel(x, W_enc, b_enc, W_dec):
    features = _encoder(x, W_enc, b_enc.reshape(1, HID))
    tbits = _thresholds_sc(
        lax.bitcast_convert_type(features, jnp.int32).reshape(-1))
    thr = lax.bitcast_convert_type(tbits, jnp.float32).reshape(N_TOK, 1)
    sparse, recon = _mask_and_decode(features, thr, W_dec)
    return (sparse, recon)


# speedup vs baseline: 1.0959x; 1.0959x over previous
import functools

import jax
import jax.numpy as jnp
from jax import lax
from jax.experimental import pallas as pl
from jax.experimental.pallas import tpu as pltpu
from jax.experimental.pallas import tpu_sc as plsc

N_TOK = 2048
IN_DIM = 768
HID = 32768
K = 64

_NC = 2
_NS = 16
_NW = _NC * _NS
_RPW = N_TOK // _NW
_NBINS = 256


_H_BLK = 1024


def _enc_body(x_ref, w_ref, b_ref, out_ref):
    acc = lax.dot_general(
        x_ref[...], w_ref[...],
        (((1,), (1,)), ((), ())),
        preferred_element_type=jnp.float32,
    )
    out_ref[...] = jnp.maximum(acc + b_ref[...], 0.0)


def _encoder(x, W_enc, b_enc2d):
    ntok = x.shape[0]
    grid = (HID // _H_BLK,)
    return pl.pallas_call(
        _enc_body,
        grid=grid,
        in_specs=[
            pl.BlockSpec((ntok, IN_DIM), lambda h: (0, 0)),
            pl.BlockSpec((_H_BLK, IN_DIM), lambda h: (h, 0)),
            pl.BlockSpec((1, _H_BLK), lambda h: (0, h)),
        ],
        out_specs=pl.BlockSpec((ntok, _H_BLK), lambda h: (0, h)),
        out_shape=jax.ShapeDtypeStruct((ntok, HID), jnp.float32),
    )(x, W_enc, b_enc2d)


_UNROLL = 4


def _zero_hist(hist_ref, ncopy):
    zeros = jnp.zeros((16,), jnp.int32)

    @plsc.parallel_loop(0, _NBINS * ncopy // 16, unroll=8)
    def _(i):
        hist_ref[pl.ds(i * 16, 16)] = zeros


def _suffix_select(hist_ref, need, ncopy):
    def body(jj, st):
        carry, cnt_ge, above = st
        j = 15 - jj
        hv = jnp.zeros((16,), jnp.int32)
        for l in range(ncopy):
            hv = hv + hist_ref[pl.ds(l * _NBINS + j * 16, 16)]
        a = jnp.flip(jnp.cumsum(jnp.flip(hv, 0)), 0) + carry
        ge = a >= need
        cnt_ge = cnt_ge + jnp.sum(ge.astype(jnp.int32))
        above = jnp.maximum(above, jnp.max(jnp.where(ge, 0, a)))
        return (carry + jnp.sum(hv), cnt_ge, above)

    z = jnp.int32(0)
    _, cnt_ge, above = lax.fori_loop(0, 16, body, (z, z, z))
    return cnt_ge - 1, above


def _hist_full(src_ref, nvec, shift, maskval, hist_ref, lane_base, ones):
    @plsc.parallel_loop(0, nvec, unroll=8)
    def _(i):
        bits = jnp.maximum(src_ref[pl.ds(i * 16, 16)], 0)
        b = (lax.shift_right_logical(bits, shift) & maskval) + (
            lane_base + (i & (_UNROLL - 1)) * (_NBINS * 16))
        plsc.addupdate_scatter(hist_ref, [b], ones)


def _hist_masked(src_ref, n, shift, maskval, hist_ref, lane_base, ones):
    iota = lax.iota(jnp.int32, 16)

    @plsc.parallel_loop(0, (n + 15) // 16, unroll=4)
    def _(i):
        bits = jnp.maximum(src_ref[pl.ds(i * 16, 16)], 0)
        b = (lax.shift_right_logical(bits, shift) & maskval) + lane_base
        lanes_ok = (i * 16 + iota) < n
        plsc.addupdate_scatter(hist_ref, [b], ones, mask=lanes_ok)


def _compress_full(src_ref, nvec, shift, maskval, bsel, dst_ref):
    @plsc.parallel_loop(0, nvec, unroll=8, carry=jnp.int32(0))
    def off(i, off):
        v = src_ref[pl.ds(i * 16, 16)]
        b = lax.shift_right_logical(jnp.maximum(v, 0), shift) & maskval
        m = b == bsel
        plsc.store_compressed(dst_ref.at[pl.ds(off, 16)], v, mask=m)
        return off + jnp.sum(m.astype(jnp.int32))

    return off


def _compress_masked(src_ref, n, shift, maskval, bsel, dst_ref):
    iota = lax.iota(jnp.int32, 16)

    @plsc.parallel_loop(0, (n + 15) // 16, unroll=4, carry=jnp.int32(0))
    def off(i, off):
        v = src_ref[pl.ds(i * 16, 16)]
        b = lax.shift_right_logical(jnp.maximum(v, 0), shift) & maskval
        m = (b == bsel) & ((i * 16 + iota) < n)
        plsc.store_compressed(dst_ref.at[pl.ds(off, 16)], v, mask=m)
        return off + jnp.sum(m.astype(jnp.int32))

    return off


def _row_threshold(row_ref, cand_ref, hist_ref, lane_base, ones):
    need = jnp.int32(K)
    _zero_hist(hist_ref, 16 * _UNROLL)
    _hist_full(row_ref, HID // 16, 23, 0xFF, hist_ref, lane_base, ones)
    b1, above = _suffix_select(hist_ref, need, 16 * _UNROLL)
    need = need - above
    tbits = b1 << 23
    ncand = _compress_full(row_ref, HID // 16, 23, 0xFF, b1, cand_ref)
    for shift, width in ((15, 8), (7, 8), (0, 7)):
        maskval = (1 << width) - 1
        _zero_hist(hist_ref, 16)
        _hist_masked(cand_ref, ncand, shift, maskval, hist_ref, lane_base, ones)
        bk, above = _suffix_select(hist_ref, need, 16)
        need = need - above
        tbits = tbits | (bk << shift)
        if shift != 0:
            ncand = _compress_masked(cand_ref, ncand, shift, maskval, bk,
                                     cand_ref)
    return tbits


def _thresholds_sc(features_flat):
    ntok = features_flat.shape[0] // HID
    rpw = ntok // _NW
    mesh = plsc.VectorSubcoreMesh(
        core_axis_name="c", subcore_axis_name="s",
        num_cores=_NC, num_subcores=_NS)

    @functools.partial(
        pl.kernel,
        out_type=jax.ShapeDtypeStruct((ntok,), jnp.int32),
        mesh=mesh,
        compiler_params=pltpu.CompilerParams(needs_layout_passes=False),
        scratch_types=[
            pltpu.VMEM((HID,), jnp.int32),
            pltpu.VMEM((HID,), jnp.int32),
            pltpu.VMEM((HID + 16,), jnp.int32),
            pltpu.VMEM((_NBINS * 16 * _UNROLL,), jnp.int32),
            pltpu.VMEM((rpw,), jnp.int32),
            pltpu.SemaphoreType.DMA,
            pltpu.SemaphoreType.DMA,
        ],
    )
    def k(feat_hbm, out_hbm, row_a, row_b, cand_v, hist_v, thr_v, sem_a, sem_b):
        wid = lax.axis_index("s") * _NC + lax.axis_index("c")
        iota = lax.iota(jnp.int32, 16)
        lane_base = iota * _NBINS
        ones = jnp.ones((16,), jnp.int32)
        base_row = wid * rpw
        ngrp = rpw // 16

        def dma(r, buf, sem):
            return pltpu.make_async_copy(
                feat_hbm.at[pl.ds((base_row + r) * HID, HID)], buf, sem)

        def merge(tvecs, r, t):
            lane = r & 15
            grp = r >> 4
            return tuple(
                jnp.where((iota == lane) & (grp == g), t, tv)
                for g, tv in enumerate(tvecs))

        dma(0, row_a, sem_a).start()

        def pair_body(p, tvecs):
            ra = 2 * p
            dma(ra, row_a, sem_a).wait()
            dma(ra + 1, row_b, sem_b).start()
            t_a = _row_threshold(row_a, cand_v, hist_v, lane_base, ones)
            tvecs = merge(tvecs, ra, t_a)
            dma(ra + 1, row_b, sem_b).wait()

            @pl.when(p < rpw // 2 - 1)
            def _():
                dma(ra + 2, row_a, sem_a).start()

            t_b = _row_threshold(row_b, cand_v, hist_v, lane_base, ones)
            return merge(tvecs, ra + 1, t_b)

        z16 = jnp.zeros((16,), jnp.int32)
        tvecs = lax.fori_loop(0, rpw // 2, pair_body,
                              tuple(z16 for _ in range(ngrp)))
        for g in range(ngrp):
            thr_v[pl.ds(g * 16, 16)] = tvecs[g]
        pltpu.sync_copy(thr_v, out_hbm.at[pl.ds(base_row, rpw)])

    return k(features_flat)


_T_BLK = 256
_HC_BLK = 2048


def _dec_body(feat_ref, thr_ref, wdec_ref, sparse_ref, recon_ref, acc_ref):
    h = pl.program_id(0)
    t = pl.program_id(1)
    nh = pl.num_programs(0)
    f = feat_ref[...]
    tv = thr_ref[...]
    s = jnp.where((f >= tv) & (f > 0.0), f, 0.0)
    sparse_ref[...] = s
    part = lax.dot_general(
        s.astype(jnp.bfloat16), wdec_ref[...].astype(jnp.bfloat16),
        (((1,), (1,)), ((), ())),
        preferred_element_type=jnp.float32,
    )
    rows = pl.ds(t * _T_BLK, _T_BLK)

    @pl.when(h == 0)
    def _():
        acc_ref[rows, :] = part

    @pl.when(h > 0)
    def _():
        acc_ref[rows, :] = acc_ref[rows, :] + part

    @pl.when(h == nh - 1)
    def _():
        recon_ref[...] = acc_ref[rows, :]


def _mask_and_decode(features, thresholds2d, W_dec):
    grid = (HID // _HC_BLK, N_TOK // _T_BLK)
    return pl.pallas_call(
        _dec_body,
        grid=grid,
        in_specs=[
            pl.BlockSpec((_T_BLK, _HC_BLK), lambda h, t: (t, h)),
            pl.BlockSpec((_T_BLK, 1), lambda h, t: (t, 0)),
            pl.BlockSpec((IN_DIM, _HC_BLK), lambda h, t: (0, h)),
        ],
        out_specs=[
            pl.BlockSpec((_T_BLK, _HC_BLK), lambda h, t: (t, h)),
            pl.BlockSpec((_T_BLK, IN_DIM), lambda h, t: (t, 0)),
        ],
        out_shape=[
            jax.ShapeDtypeStruct((N_TOK, HID), jnp.float32),
            jax.ShapeDtypeStruct((N_TOK, IN_DIM), jnp.float32),
        ],
        scratch_shapes=[pltpu.VMEM((N_TOK, IN_DIM), jnp.float32)],
    )(features, thresholds2d, W_dec)


def kernel(x, W_enc, b_enc, W_dec):
    features = _encoder(x, W_enc, b_enc.reshape(1, HID))
    tbits = _thresholds_sc(
        lax.bitcast_convert_type(features, jnp.int32).reshape(-1))
    thr = lax.bitcast_convert_type(tbits, jnp.float32).reshape(N_TOK, 1)
    sparse, recon = _mask_and_decode(features, thr, W_dec)
    return (sparse, recon)


# speedup vs baseline: 1.1311x; 1.0321x over previous
import functools

import jax
import jax.numpy as jnp
from jax import lax
from jax.experimental import pallas as pl
from jax.experimental.pallas import tpu as pltpu
from jax.experimental.pallas import tpu_sc as plsc

N_TOK = 2048
IN_DIM = 768
HID = 32768
K = 64

_NC = 2
_NS = 16
_NW = _NC * _NS
_RPW = N_TOK // _NW
_NBINS = 256


_H_BLK = 1024


def _enc_body(x_ref, w_ref, b_ref, out_ref):
    acc = lax.dot_general(
        x_ref[...], w_ref[...],
        (((1,), (1,)), ((), ())),
        preferred_element_type=jnp.float32,
    )
    out_ref[...] = jnp.maximum(acc + b_ref[...], 0.0)


def _encoder(x, W_enc, b_enc2d):
    ntok = x.shape[0]
    grid = (HID // _H_BLK,)
    return pl.pallas_call(
        _enc_body,
        grid=grid,
        in_specs=[
            pl.BlockSpec((ntok, IN_DIM), lambda h: (0, 0)),
            pl.BlockSpec((_H_BLK, IN_DIM), lambda h: (h, 0)),
            pl.BlockSpec((1, _H_BLK), lambda h: (0, h)),
        ],
        out_specs=pl.BlockSpec((ntok, _H_BLK), lambda h: (0, h)),
        out_shape=jax.ShapeDtypeStruct((ntok, HID), jnp.float32),
    )(x, W_enc, b_enc2d)


_UNROLL = 4


def _zero_hist(hist_ref, ncopy):
    zeros = jnp.zeros((16,), jnp.int32)

    @plsc.parallel_loop(0, _NBINS * ncopy // 16, unroll=8)
    def _(i):
        hist_ref[pl.ds(i * 16, 16)] = zeros


def _suffix_select(hist_ref, need, ncopy):
    zeros = jnp.zeros((16,), jnp.int32)

    def body(jj, st):
        carry, cnt_ge, above = st
        j = 15 - jj
        hv = jnp.zeros((16,), jnp.int32)
        for l in range(ncopy):
            hv = hv + hist_ref[pl.ds(l * _NBINS + j * 16, 16)]
            hist_ref[pl.ds(l * _NBINS + j * 16, 16)] = zeros
        a = jnp.flip(jnp.cumsum(jnp.flip(hv, 0)), 0) + carry
        ge = a >= need
        cnt_ge = cnt_ge + jnp.sum(ge.astype(jnp.int32))
        above = jnp.maximum(above, jnp.max(jnp.where(ge, 0, a)))
        return (carry + jnp.sum(hv), cnt_ge, above)

    z = jnp.int32(0)
    _, cnt_ge, above = lax.fori_loop(0, 16, body, (z, z, z))
    return cnt_ge - 1, above


def _hist_full(src_ref, nvec, shift, maskval, hist_ref, lane_base, ones):
    @plsc.parallel_loop(0, nvec, unroll=8)
    def _(i):
        bits = jnp.maximum(src_ref[pl.ds(i * 16, 16)], 0)
        b = (lax.shift_right_logical(bits, shift) & maskval) + (
            lane_base + (i & (_UNROLL - 1)) * (_NBINS * 16))
        plsc.addupdate_scatter(hist_ref, [b], ones)


def _hist_masked(src_ref, n, shift, maskval, hist_ref, lane_base, ones):
    iota = lax.iota(jnp.int32, 16)

    @plsc.parallel_loop(0, (n + 15) // 16, unroll=4)
    def _(i):
        bits = jnp.maximum(src_ref[pl.ds(i * 16, 16)], 0)
        b = (lax.shift_right_logical(bits, shift) & maskval) + lane_base
        lanes_ok = (i * 16 + iota) < n
        plsc.addupdate_scatter(hist_ref, [b], ones, mask=lanes_ok)


def _compress_full(src_ref, nvec, shift, maskval, bsel, dst_ref):
    @plsc.parallel_loop(0, nvec, unroll=8, carry=jnp.int32(0))
    def off(i, off):
        v = src_ref[pl.ds(i * 16, 16)]
        b = lax.shift_right_logical(jnp.maximum(v, 0), shift) & maskval
        m = b == bsel
        plsc.store_compressed(dst_ref.at[pl.ds(off, 16)], v, mask=m)
        return off + jnp.sum(m.astype(jnp.int32))

    return off


def _compress_masked(src_ref, n, shift, maskval, bsel, dst_ref):
    iota = lax.iota(jnp.int32, 16)

    @plsc.parallel_loop(0, (n + 15) // 16, unroll=4, carry=jnp.int32(0))
    def off(i, off):
        v = src_ref[pl.ds(i * 16, 16)]
        b = lax.shift_right_logical(jnp.maximum(v, 0), shift) & maskval
        m = (b == bsel) & ((i * 16 + iota) < n)
        plsc.store_compressed(dst_ref.at[pl.ds(off, 16)], v, mask=m)
        return off + jnp.sum(m.astype(jnp.int32))

    return off


def _row_threshold(row_ref, cand_ref, hist_ref, lane_base, ones):
    need = jnp.int32(K)
    _hist_full(row_ref, HID // 16, 23, 0xFF, hist_ref, lane_base, ones)
    b1, above = _suffix_select(hist_ref, need, 16 * _UNROLL)
    need = need - above
    tbits = b1 << 23
    ncand = _compress_full(row_ref, HID // 16, 23, 0xFF, b1, cand_ref)
    for shift, width in ((15, 8), (7, 8), (0, 7)):
        maskval = (1 << width) - 1
        _hist_masked(cand_ref, ncand, shift, maskval, hist_ref, lane_base, ones)
        bk, above = _suffix_select(hist_ref, need, 16)
        need = need - above
        tbits = tbits | (bk << shift)
        if shift != 0:
            ncand = _compress_masked(cand_ref, ncand, shift, maskval, bk,
                                     cand_ref)
    return tbits


def _thresholds_sc(features_flat):
    ntok = features_flat.shape[0] // HID
    rpw = ntok // _NW
    mesh = plsc.VectorSubcoreMesh(
        core_axis_name="c", subcore_axis_name="s",
        num_cores=_NC, num_subcores=_NS)

    @functools.partial(
        pl.kernel,
        out_type=jax.ShapeDtypeStruct((ntok,), jnp.int32),
        mesh=mesh,
        compiler_params=pltpu.CompilerParams(needs_layout_passes=False),
        scratch_types=[
            pltpu.VMEM((HID,), jnp.int32),
            pltpu.VMEM((HID,), jnp.int32),
            pltpu.VMEM((HID + 16,), jnp.int32),
            pltpu.VMEM((_NBINS * 16 * _UNROLL,), jnp.int32),
            pltpu.VMEM((rpw,), jnp.int32),
            pltpu.SemaphoreType.DMA,
            pltpu.SemaphoreType.DMA,
        ],
    )
    def k(feat_hbm, out_hbm, row_a, row_b, cand_v, hist_v, thr_v, sem_a, sem_b):
        wid = lax.axis_index("s") * _NC + lax.axis_index("c")
        iota = lax.iota(jnp.int32, 16)
        lane_base = iota * _NBINS
        ones = jnp.ones((16,), jnp.int32)
        base_row = wid * rpw
        ngrp = rpw // 16
        _zero_hist(hist_v, 16 * _UNROLL)

        def dma(r, buf, sem):
            return pltpu.make_async_copy(
                feat_hbm.at[pl.ds((base_row + r) * HID, HID)], buf, sem)

        def merge(tvecs, r, t):
            lane = r & 15
            grp = r >> 4
            return tuple(
                jnp.where((iota == lane) & (grp == g), t, tv)
                for g, tv in enumerate(tvecs))

        dma(0, row_a, sem_a).start()

        def pair_body(p, tvecs):
            ra = 2 * p
            dma(ra, row_a, sem_a).wait()
            dma(ra + 1, row_b, sem_b).start()
            t_a = _row_threshold(row_a, cand_v, hist_v, lane_base, ones)
            tvecs = merge(tvecs, ra, t_a)
            dma(ra + 1, row_b, sem_b).wait()

            @pl.when(p < rpw // 2 - 1)
            def _():
                dma(ra + 2, row_a, sem_a).start()

            t_b = _row_threshold(row_b, cand_v, hist_v, lane_base, ones)
            return merge(tvecs, ra + 1, t_b)

        z16 = jnp.zeros((16,), jnp.int32)
        tvecs = lax.fori_loop(0, rpw // 2, pair_body,
                              tuple(z16 for _ in range(ngrp)))
        for g in range(ngrp):
            thr_v[pl.ds(g * 16, 16)] = tvecs[g]
        pltpu.sync_copy(thr_v, out_hbm.at[pl.ds(base_row, rpw)])

    return k(features_flat)


_T_BLK = 256
_HC_BLK = 2048


def _dec_body(feat_ref, thr_ref, wdec_ref, sparse_ref, recon_ref, acc_ref):
    h = pl.program_id(0)
    t = pl.program_id(1)
    nh = pl.num_programs(0)
    f = feat_ref[...]
    tv = thr_ref[...]
    s = jnp.where((f >= tv) & (f > 0.0), f, 0.0)
    sparse_ref[...] = s
    part = lax.dot_general(
        s.astype(jnp.bfloat16), wdec_ref[...].astype(jnp.bfloat16),
        (((1,), (1,)), ((), ())),
        preferred_element_type=jnp.float32,
    )
    rows = pl.ds(t * _T_BLK, _T_BLK)

    @pl.when(h == 0)
    def _():
        acc_ref[rows, :] = part

    @pl.when(h > 0)
    def _():
        acc_ref[rows, :] = acc_ref[rows, :] + part

    @pl.when(h == nh - 1)
    def _():
        recon_ref[...] = acc_ref[rows, :]


def _mask_and_decode(features, thresholds2d, W_dec):
    grid = (HID // _HC_BLK, N_TOK // _T_BLK)
    return pl.pallas_call(
        _dec_body,
        grid=grid,
        in_specs=[
            pl.BlockSpec((_T_BLK, _HC_BLK), lambda h, t: (t, h)),
            pl.BlockSpec((_T_BLK, 1), lambda h, t: (t, 0)),
            pl.BlockSpec((IN_DIM, _HC_BLK), lambda h, t: (0, h)),
        ],
        out_specs=[
            pl.BlockSpec((_T_BLK, _HC_BLK), lambda h, t: (t, h)),
            pl.BlockSpec((_T_BLK, IN_DIM), lambda h, t: (t, 0)),
        ],
        out_shape=[
            jax.ShapeDtypeStruct((N_TOK, HID), jnp.float32),
            jax.ShapeDtypeStruct((N_TOK, IN_DIM), jnp.float32),
        ],
        scratch_shapes=[pltpu.VMEM((N_TOK, IN_DIM), jnp.float32)],
    )(features, thresholds2d, W_dec)


def kernel(x, W_enc, b_enc, W_dec):
    features = _encoder(x, W_enc, b_enc.reshape(1, HID))
    tbits = _thresholds_sc(
        lax.bitcast_convert_type(features, jnp.int32).reshape(-1))
    thr = lax.bitcast_convert_type(tbits, jnp.float32).reshape(N_TOK, 1)
    sparse, recon = _mask_and_decode(features, thr, W_dec)
    return (sparse, recon)


# speedup vs baseline: 1.1430x; 1.0105x over previous
import functools

import jax
import jax.numpy as jnp
from jax import lax
from jax.experimental import pallas as pl
from jax.experimental.pallas import tpu as pltpu
from jax.experimental.pallas import tpu_sc as plsc

N_TOK = 2048
IN_DIM = 768
HID = 32768
K = 64

_NC = 2
_NS = 16
_NW = _NC * _NS
_RPW = N_TOK // _NW
_NBINS = 256


_H_BLK = 1024


def _enc_body(x_ref, w_ref, b_ref, out_ref):
    acc = lax.dot_general(
        x_ref[...], w_ref[...],
        (((1,), (1,)), ((), ())),
        preferred_element_type=jnp.float32,
    )
    out_ref[...] = jnp.maximum(acc + b_ref[...], 0.0)


def _encoder(x, W_enc, b_enc2d):
    ntok = x.shape[0]
    grid = (HID // _H_BLK,)
    return pl.pallas_call(
        _enc_body,
        grid=grid,
        in_specs=[
            pl.BlockSpec((ntok, IN_DIM), lambda h: (0, 0)),
            pl.BlockSpec((_H_BLK, IN_DIM), lambda h: (h, 0)),
            pl.BlockSpec((1, _H_BLK), lambda h: (0, h)),
        ],
        out_specs=pl.BlockSpec((ntok, _H_BLK), lambda h: (0, h)),
        out_shape=jax.ShapeDtypeStruct((ntok, HID), jnp.float32),
    )(x, W_enc, b_enc2d)


_UNROLL = 2


def _zero_hist(hist_ref, ncopy):
    zeros = jnp.zeros((16,), jnp.int32)

    @plsc.parallel_loop(0, _NBINS * ncopy // 16, unroll=8)
    def _(i):
        hist_ref[pl.ds(i * 16, 16)] = zeros


def _suffix_select(hist_ref, need, ncopy):
    zeros = jnp.zeros((16,), jnp.int32)

    def body(jj, st):
        carry, cnt_ge, above = st
        j = 15 - jj
        hv = jnp.zeros((16,), jnp.int32)
        for l in range(ncopy):
            hv = hv + hist_ref[pl.ds(l * _NBINS + j * 16, 16)]
            hist_ref[pl.ds(l * _NBINS + j * 16, 16)] = zeros
        a = jnp.flip(jnp.cumsum(jnp.flip(hv, 0)), 0) + carry
        ge = a >= need
        cnt_ge = cnt_ge + jnp.sum(ge.astype(jnp.int32))
        above = jnp.maximum(above, jnp.max(jnp.where(ge, 0, a)))
        return (carry + jnp.sum(hv), cnt_ge, above)

    z = jnp.int32(0)
    _, cnt_ge, above = lax.fori_loop(0, 16, body, (z, z, z))
    return cnt_ge - 1, above


def _hist_full(src_ref, nvec, shift, maskval, hist_ref, lane_base, ones):
    @plsc.parallel_loop(0, nvec, unroll=8)
    def _(i):
        bits = jnp.maximum(src_ref[pl.ds(i * 16, 16)], 0)
        b = (lax.shift_right_logical(bits, shift) & maskval) + (
            lane_base + (i & (_UNROLL - 1)) * (_NBINS * 16))
        plsc.addupdate_scatter(hist_ref, [b], ones)


def _hist_masked(src_ref, n, shift, maskval, hist_ref, lane_base, ones):
    iota = lax.iota(jnp.int32, 16)

    @plsc.parallel_loop(0, (n + 15) // 16, unroll=4)
    def _(i):
        bits = jnp.maximum(src_ref[pl.ds(i * 16, 16)], 0)
        b = (lax.shift_right_logical(bits, shift) & maskval) + lane_base
        lanes_ok = (i * 16 + iota) < n
        plsc.addupdate_scatter(hist_ref, [b], ones, mask=lanes_ok)


def _compress_full(src_ref, nvec, shift, maskval, bsel, dst_ref):
    @plsc.parallel_loop(0, nvec, unroll=8, carry=jnp.int32(0))
    def off(i, off):
        v = src_ref[pl.ds(i * 16, 16)]
        b = lax.shift_right_logical(jnp.maximum(v, 0), shift) & maskval
        m = b == bsel
        plsc.store_compressed(dst_ref.at[pl.ds(off, 16)], v, mask=m)
        return off + jnp.sum(m.astype(jnp.int32))

    return off


def _compress_masked(src_ref, n, shift, maskval, bsel, dst_ref):
    iota = lax.iota(jnp.int32, 16)

    @plsc.parallel_loop(0, (n + 15) // 16, unroll=4, carry=jnp.int32(0))
    def off(i, off):
        v = src_ref[pl.ds(i * 16, 16)]
        b = lax.shift_right_logical(jnp.maximum(v, 0), shift) & maskval
        m = (b == bsel) & ((i * 16 + iota) < n)
        plsc.store_compressed(dst_ref.at[pl.ds(off, 16)], v, mask=m)
        return off + jnp.sum(m.astype(jnp.int32))

    return off


def _row_threshold(row_ref, cand_ref, hist_ref, lane_base, ones):
    need = jnp.int32(K)
    _hist_full(row_ref, HID // 16, 23, 0xFF, hist_ref, lane_base, ones)
    b1, above = _suffix_select(hist_ref, need, 16 * _UNROLL)
    need = need - above
    tbits = b1 << 23
    ncand = _compress_full(row_ref, HID // 16, 23, 0xFF, b1, cand_ref)
    for shift, width in ((15, 8), (7, 8), (0, 7)):
        maskval = (1 << width) - 1
        _hist_masked(cand_ref, ncand, shift, maskval, hist_ref, lane_base, ones)
        bk, above = _suffix_select(hist_ref, need, 16)
        need = need - above
        tbits = tbits | (bk << shift)
        if shift != 0:
            ncand = _compress_masked(cand_ref, ncand, shift, maskval, bk,
                                     cand_ref)
    return tbits


def _thresholds_sc(features_flat):
    ntok = features_flat.shape[0] // HID
    rpw = ntok // _NW
    mesh = plsc.VectorSubcoreMesh(
        core_axis_name="c", subcore_axis_name="s",
        num_cores=_NC, num_subcores=_NS)

    @functools.partial(
        pl.kernel,
        out_type=jax.ShapeDtypeStruct((ntok,), jnp.int32),
        mesh=mesh,
        compiler_params=pltpu.CompilerParams(needs_layout_passes=False),
        scratch_types=[
            pltpu.VMEM((HID,), jnp.int32),
            pltpu.VMEM((HID,), jnp.int32),
            pltpu.VMEM((HID + 16,), jnp.int32),
            pltpu.VMEM((_NBINS * 16 * _UNROLL,), jnp.int32),
            pltpu.VMEM((rpw,), jnp.int32),
            pltpu.SemaphoreType.DMA,
            pltpu.SemaphoreType.DMA,
        ],
    )
    def k(feat_hbm, out_hbm, row_a, row_b, cand_v, hist_v, thr_v, sem_a, sem_b):
        wid = lax.axis_index("s") * _NC + lax.axis_index("c")
        iota = lax.iota(jnp.int32, 16)
        lane_base = iota * _NBINS
        ones = jnp.ones((16,), jnp.int32)
        base_row = wid * rpw
        ngrp = rpw // 16
        _zero_hist(hist_v, 16 * _UNROLL)

        def dma(r, buf, sem):
            return pltpu.make_async_copy(
                feat_hbm.at[pl.ds((base_row + r) * HID, HID)], buf, sem)

        def merge(tvecs, r, t):
            lane = r & 15
            grp = r >> 4
            return tuple(
                jnp.where((iota == lane) & (grp == g), t, tv)
                for g, tv in enumerate(tvecs))

        dma(0, row_a, sem_a).start()

        def pair_body(p, tvecs):
            ra = 2 * p
            dma(ra, row_a, sem_a).wait()
            dma(ra + 1, row_b, sem_b).start()
            t_a = _row_threshold(row_a, cand_v, hist_v, lane_base, ones)
            tvecs = merge(tvecs, ra, t_a)
            dma(ra + 1, row_b, sem_b).wait()

            @pl.when(p < rpw // 2 - 1)
            def _():
                dma(ra + 2, row_a, sem_a).start()

            t_b = _row_threshold(row_b, cand_v, hist_v, lane_base, ones)
            return merge(tvecs, ra + 1, t_b)

        z16 = jnp.zeros((16,), jnp.int32)
        tvecs = lax.fori_loop(0, rpw // 2, pair_body,
                              tuple(z16 for _ in range(ngrp)))
        for g in range(ngrp):
            thr_v[pl.ds(g * 16, 16)] = tvecs[g]
        pltpu.sync_copy(thr_v, out_hbm.at[pl.ds(base_row, rpw)])

    return k(features_flat)


_T_BLK = 256
_HC_BLK = 2048


def _dec_body(feat_ref, thr_ref, wdec_ref, sparse_ref, recon_ref, acc_ref):
    h = pl.program_id(0)
    t = pl.program_id(1)
    nh = pl.num_programs(0)
    f = feat_ref[...]
    tv = thr_ref[...]
    s = jnp.where((f >= tv) & (f > 0.0), f, 0.0)
    sparse_ref[...] = s
    part = lax.dot_general(
        s.astype(jnp.bfloat16), wdec_ref[...].astype(jnp.bfloat16),
        (((1,), (1,)), ((), ())),
        preferred_element_type=jnp.float32,
    )
    rows = pl.ds(t * _T_BLK, _T_BLK)

    @pl.when(h == 0)
    def _():
        acc_ref[rows, :] = part

    @pl.when(h > 0)
    def _():
        acc_ref[rows, :] = acc_ref[rows, :] + part

    @pl.when(h == nh - 1)
    def _():
        recon_ref[...] = acc_ref[rows, :]


def _mask_and_decode(features, thresholds2d, W_dec):
    grid = (HID // _HC_BLK, N_TOK // _T_BLK)
    return pl.pallas_call(
        _dec_body,
        grid=grid,
        in_specs=[
            pl.BlockSpec((_T_BLK, _HC_BLK), lambda h, t: (t, h)),
            pl.BlockSpec((_T_BLK, 1), lambda h, t: (t, 0)),
            pl.BlockSpec((IN_DIM, _HC_BLK), lambda h, t: (0, h)),
        ],
        out_specs=[
            pl.BlockSpec((_T_BLK, _HC_BLK), lambda h, t: (t, h)),
            pl.BlockSpec((_T_BLK, IN_DIM), lambda h, t: (t, 0)),
        ],
        out_shape=[
            jax.ShapeDtypeStruct((N_TOK, HID), jnp.float32),
            jax.ShapeDtypeStruct((N_TOK, IN_DIM), jnp.float32),
        ],
        scratch_shapes=[pltpu.VMEM((N_TOK, IN_DIM), jnp.float32)],
    )(features, thresholds2d, W_dec)


def kernel(x, W_enc, b_enc, W_dec):
    features = _encoder(x, W_enc, b_enc.reshape(1, HID))
    tbits = _thresholds_sc(
        lax.bitcast_convert_type(features, jnp.int32).reshape(-1))
    thr = lax.bitcast_convert_type(tbits, jnp.float32).reshape(N_TOK, 1)
    sparse, recon = _mask_and_decode(features, thr, W_dec)
    return (sparse, recon)
